# speedup vs baseline: 5.2164x; 5.2164x over previous
import functools

import jax
import jax.numpy as jnp
from jax import lax
from jax.experimental import pallas as pl
from jax.experimental.pallas import tpu as pltpu
from jax.experimental.pallas import tpu_sc as plsc

NC = 2
NS = 16
NW = NC * NS
LANES = 16
CH = 128


def _prep_body(seg_ids_ref, pos_ref, seg_tab_ref, comb_ref, cidx_ref):
    comb_ref[...] = pos_ref[...][:, None, :] + seg_tab_ref[...][None, :, :]
    s_iota = lax.broadcasted_iota(jnp.int32, seg_ids_ref.shape, 1)
    cidx_ref[...] = seg_ids_ref[...] + 3 * s_iota


def _sc_body(n_chunks, tok_hbm, ids_hbm, cidx_hbm, comb_hbm, out_hbm,
             idx_t, idx_c, rows_t, rows_c, sem_t, sem_c):
    wid = lax.axis_index("s") * NC + lax.axis_index("c")
    base0 = wid * (n_chunks * CH)

    def chunk_body(g, carry):
        base = base0 + g * CH
        pltpu.sync_copy(ids_hbm.at[pl.ds(base, CH)], idx_t)
        pltpu.sync_copy(cidx_hbm.at[pl.ds(base, CH)], idx_c)
        ct = pltpu.async_copy(tok_hbm.at[idx_t], rows_t, sem_t)
        cc = pltpu.async_copy(comb_hbm.at[idx_c], rows_c, sem_c)
        ct.wait()
        cc.wait()

        def row_body(r, rcarry):
            for c in range(8):
                sl = pl.ds(c * LANES, LANES)
                rows_t[r, sl] = rows_t[r, sl] + rows_c[r, sl]
            return rcarry

        lax.fori_loop(0, CH, row_body, 0)
        pltpu.sync_copy(rows_t, out_hbm.at[pl.ds(base, CH)])
        return carry

    lax.fori_loop(0, n_chunks, chunk_body, 0)


def kernel(input_ids, segment_ids, token_table, segment_table, pos_table):
    B, S = input_ids.shape
    H = token_table.shape[1]
    R = B * S
    assert R % (NW * CH) == 0
    n_chunks = R // (NW * CH)

    comb3, cidx = pl.pallas_call(
        _prep_body,
        out_shape=(
            jax.ShapeDtypeStruct((S, 3, H), jnp.float32),
            jax.ShapeDtypeStruct((B, S), jnp.int32),
        ),
    )(segment_ids.astype(jnp.int32), pos_table[:S], segment_table)

    comb = comb3.reshape(S * 3, H)
    ids_flat = input_ids.astype(jnp.int32).reshape(R)
    cidx_flat = cidx.reshape(R)

    sc_fn = functools.partial(
        pl.kernel,
        out_type=jax.ShapeDtypeStruct((R, H), jnp.float32),
        mesh=plsc.VectorSubcoreMesh(core_axis_name="c", subcore_axis_name="s"),
        scratch_types=[
            pltpu.VMEM((CH,), jnp.int32),
            pltpu.VMEM((CH,), jnp.int32),
            pltpu.VMEM((CH, H), jnp.float32),
            pltpu.VMEM((CH, H), jnp.float32),
            pltpu.SemaphoreType.DMA,
            pltpu.SemaphoreType.DMA,
        ],
    )(_sc_body_with_chunks(n_chunks))

    out2d = sc_fn(token_table, ids_flat, cidx_flat, comb)
    return out2d.reshape(B, S, H)


def _sc_body_with_chunks(n_chunks):
    return functools.partial(_sc_body, n_chunks)


# speedup vs baseline: 7.6801x; 1.4723x over previous
import functools

import jax
import jax.numpy as jnp
from jax import lax
from jax.experimental import pallas as pl
from jax.experimental.pallas import tpu as pltpu
from jax.experimental.pallas import tpu_sc as plsc

NC = 2
NS = 16
NW = NC * NS
LANES = 16
CH = 128


def _prep_body(seg_ids_ref, pos_ref, seg_tab_ref, comb_ref, cidx_ref):
    comb_ref[...] = pos_ref[...][:, None, :] + seg_tab_ref[...][None, :, :]
    s_iota = lax.broadcasted_iota(jnp.int32, seg_ids_ref.shape, 1)
    cidx_ref[...] = seg_ids_ref[...] + 3 * s_iota


def _sc_body(n_chunks, tok_hbm, ids_hbm, cidx_hbm, comb_hbm, out_hbm,
             idx_t, idx_c, rt0, rt1, ro0, ro1, gs0, gs1, ss0, ss1):
    wid = lax.axis_index("s") * NC + lax.axis_index("c")
    pw = n_chunks * CH
    base0 = wid * pw

    pltpu.sync_copy(ids_hbm.at[pl.ds(base0, pw)], idx_t)
    pltpu.sync_copy(cidx_hbm.at[pl.ds(base0, pw)], idx_c)

    def fire(g, rt, ro, gsem):
        it = idx_t.at[pl.ds(g * CH, CH)]
        ic = idx_c.at[pl.ds(g * CH, CH)]
        pltpu.async_copy(tok_hbm.at[it], rt, gsem)
        pltpu.async_copy(comb_hbm.at[ic], ro, gsem)

    def wait_scatter(ro, ssem):
        pltpu.make_async_copy(ro, out_hbm.at[pl.ds(base0, CH)], ssem).wait()

    def proc(g, rt, ro, gsem, ssem):
        dummy = tok_hbm.at[pl.ds(0, CH)]
        pltpu.make_async_copy(dummy, rt, gsem).wait()
        pltpu.make_async_copy(dummy, ro, gsem).wait()

        def row_body(r, rcarry):
            for c in range(8):
                sl = pl.ds(c * LANES, LANES)
                ro[r, sl] = ro[r, sl] + rt[r, sl]
            return rcarry

        lax.fori_loop(0, CH, row_body, 0)

        pltpu.async_copy(ro, out_hbm.at[pl.ds(base0 + g * CH, CH)], ssem)

    n_pairs = n_chunks // 2
    fire(0, rt0, ro0, gs0)
    fire(1, rt1, ro1, gs1)

    def pair_body(k, carry):
        g0 = 2 * k
        proc(g0, rt0, ro0, gs0, ss0)

        @pl.when(k < n_pairs - 1)
        def _():
            wait_scatter(ro0, ss0)
            fire(g0 + 2, rt0, ro0, gs0)

        proc(g0 + 1, rt1, ro1, gs1, ss1)

        @pl.when(k < n_pairs - 1)
        def _():
            wait_scatter(ro1, ss1)
            fire(g0 + 3, rt1, ro1, gs1)

        return carry

    lax.fori_loop(0, n_pairs, pair_body, 0)
    wait_scatter(ro0, ss0)
    wait_scatter(ro1, ss1)


def kernel(input_ids, segment_ids, token_table, segment_table, pos_table):
    B, S = input_ids.shape
    H = token_table.shape[1]
    R = B * S
    assert R % (NW * CH * 2) == 0
    n_chunks = R // (NW * CH)

    comb3, cidx = pl.pallas_call(
        _prep_body,
        out_shape=(
            jax.ShapeDtypeStruct((S, 3, H), jnp.float32),
            jax.ShapeDtypeStruct((B, S), jnp.int32),
        ),
    )(segment_ids.astype(jnp.int32), pos_table[:S], segment_table)

    comb = comb3.reshape(S * 3, H)
    ids_flat = input_ids.astype(jnp.int32).reshape(R)
    cidx_flat = cidx.reshape(R)

    sc_fn = functools.partial(
        pl.kernel,
        out_type=jax.ShapeDtypeStruct((R, H), jnp.float32),
        mesh=plsc.VectorSubcoreMesh(core_axis_name="c", subcore_axis_name="s"),
        scratch_types=[
            pltpu.VMEM((R // NW,), jnp.int32),
            pltpu.VMEM((R // NW,), jnp.int32),
            pltpu.VMEM((CH, H), jnp.float32),
            pltpu.VMEM((CH, H), jnp.float32),
            pltpu.VMEM((CH, H), jnp.float32),
            pltpu.VMEM((CH, H), jnp.float32),
            pltpu.SemaphoreType.DMA,
            pltpu.SemaphoreType.DMA,
            pltpu.SemaphoreType.DMA,
            pltpu.SemaphoreType.DMA,
        ],
    )(functools.partial(_sc_body, n_chunks))

    out2d = sc_fn(token_table, ids_flat, cidx_flat, comb)
    return out2d.reshape(B, S, H)
